# baseline (speedup 1.0000x reference)
import functools

import jax
import jax.numpy as jnp
from jax import lax
from jax.experimental import pallas as pl
from jax.experimental.pallas import tpu as pltpu
from jax.experimental.pallas import tpu_sc as plsc

_B, _N, _CIN = 8, 4096, 64
_S, _K = 512, 32
_RADIUS = 0.2
_SB = 256
_CNT = _B * _S * _K


def _fps_body(x_ref, y_ref, z_ref, cx_ref, cy_ref, cz_ref):
    x = x_ref[...]
    y = y_ref[...]
    z = z_ref[...]
    lanes = lax.broadcasted_iota(jnp.int32, (_B, _N), 1)
    slane = lax.broadcasted_iota(jnp.int32, (_B, _S), 1)

    def step(i, carry):
        dists, far, cxs, cys, czs = carry
        m = lanes == far
        cx = jnp.sum(jnp.where(m, x, 0.0), axis=1, keepdims=True)
        cy = jnp.sum(jnp.where(m, y, 0.0), axis=1, keepdims=True)
        cz = jnp.sum(jnp.where(m, z, 0.0), axis=1, keepdims=True)
        sm = slane == i
        cxs = jnp.where(sm, cx, cxs)
        cys = jnp.where(sm, cy, cys)
        czs = jnp.where(sm, cz, czs)
        d = (x - cx) ** 2 + (y - cy) ** 2 + (z - cz) ** 2
        dists = jnp.minimum(dists, d)
        mx = jnp.max(dists, axis=1, keepdims=True)
        far = jnp.min(jnp.where(dists == mx, lanes, _N), axis=1, keepdims=True)
        return dists, far, cxs, cys, czs

    zc = jnp.zeros((_B, _S), jnp.float32)
    init = (jnp.full((_B, _N), 1e10, jnp.float32), jnp.zeros((_B, 1), jnp.int32),
            zc, zc, zc)
    _, _, cxs, cys, czs = lax.fori_loop(0, _S, step, init)
    cx_ref[...] = cxs
    cy_ref[...] = cys
    cz_ref[...] = czs


def _fps(coords):
    x = coords[:, 0, :]
    y = coords[:, 1, :]
    z = coords[:, 2, :]
    cx, cy, cz = pl.pallas_call(
        _fps_body,
        out_shape=[jax.ShapeDtypeStruct((_B, _S), jnp.float32)] * 3,
    )(x, y, z)
    return jnp.stack([cx, cy, cz], axis=1)


def _ballq_body(c_ref, p_ref, o_ref):
    c = c_ref[0]
    p = p_ref[0]
    cx, cy, cz = c[:, 0:1], c[:, 1:2], c[:, 2:3]
    x, y, z = p[0:1], p[1:2], p[2:3]
    cn = (cx * cx + cy * cy) + cz * cz
    pn = (x * x + y * y) + z * z
    dp = jnp.dot(c.astype(jnp.bfloat16), p.astype(jnp.bfloat16),
                 preferred_element_type=jnp.float32)
    thr = (cn - _RADIUS * _RADIUS) + pn
    mask = (2.0 * dp >= thr).astype(jnp.int32)
    w = jnp.zeros((_SB, 128), jnp.int32)
    for j in range(_N // 128):
        w = w | (mask[:, j * 128:(j + 1) * 128] << j)
    l128 = lax.broadcasted_iota(jnp.int32, (_SB, 128), 1)
    cols = []
    for _ in range(_K):
        t = w & (-w)
        jj = lax.population_count(t - 1)
        val = jnp.where(w != 0, jj * 128 + l128, _N)
        nmin = jnp.min(val, axis=1, keepdims=True)
        cols.append(nmin)
        sh = jnp.minimum(nmin >> 7, 31)
        bit = jnp.where(nmin < _N, jnp.int32(1) << sh, 0)
        w = jnp.where(l128 == (nmin & 127), w & ~bit, w)
    nidx = jnp.concatenate(cols, axis=1)
    first = nidx[:, 0:1]
    o_ref[0] = jnp.where(nidx == _N, first, nidx)


def _ball_query(c_bsc, coords):
    grid = (_B, _S // _SB)
    return pl.pallas_call(
        _ballq_body,
        grid=grid,
        in_specs=[
            pl.BlockSpec((1, _SB, 3), lambda b, s: (b, s, 0)),
            pl.BlockSpec((1, 3, _N), lambda b, s: (b, 0, 0)),
        ],
        out_specs=pl.BlockSpec((1, _SB, _K), lambda b, s: (b, s, 0)),
        out_shape=jax.ShapeDtypeStruct((_B, _S, _K), jnp.int32),
    )(c_bsc, coords)


def _prep_body(ft_ref, pt_ref, c_ref, wf_ref, wc_ref, b1_ref, yt_ref, zt_ref):
    ft = ft_ref[0]
    pt = pt_ref[0]
    c = c_ref[0]
    wf = wf_ref[...]
    wc = wc_ref[...]
    b1 = b1_ref[...]
    yt = jnp.dot(ft, wf, preferred_element_type=jnp.float32)
    yt = yt + jnp.dot(pt, wc, preferred_element_type=jnp.float32)
    yt_ref[0] = yt + b1
    zt_ref[0] = jnp.dot(c, wc, preferred_element_type=jnp.float32)


def _prep(ft, pt, c_bsc, W1, b1):
    wf = jnp.transpose(W1[:, 3:], (1, 0))
    wc = jnp.transpose(W1[:, :3], (1, 0))
    b1r = b1[None, :]
    return pl.pallas_call(
        _prep_body,
        grid=(_B,),
        in_specs=[
            pl.BlockSpec((1, _N, _CIN), lambda b: (b, 0, 0)),
            pl.BlockSpec((1, _N, 3), lambda b: (b, 0, 0)),
            pl.BlockSpec((1, _S, 3), lambda b: (b, 0, 0)),
            pl.BlockSpec((_CIN, 64), lambda b: (0, 0)),
            pl.BlockSpec((3, 64), lambda b: (0, 0)),
            pl.BlockSpec((1, 64), lambda b: (0, 0)),
        ],
        out_specs=[
            pl.BlockSpec((1, _N, 64), lambda b: (b, 0, 0)),
            pl.BlockSpec((1, _S, 64), lambda b: (b, 0, 0)),
        ],
        out_shape=[
            jax.ShapeDtypeStruct((_B, _N, 64), jnp.float32),
            jax.ShapeDtypeStruct((_B, _S, 64), jnp.float32),
        ],
    )(ft, pt, c_bsc, wf, wc, b1r)


_NC, _NS = 2, 16
_NW = _NC * _NS
_BPW = _CNT // _NW
_CH = 512


def _sc_gather_body(table_hbm, idx_hbm, out_hbm, idx_v, rows_v, sem):
    wid = lax.axis_index("s") * _NC + lax.axis_index("c")

    def body(c, carry):
        base = wid * _BPW + c * _CH
        pltpu.sync_copy(idx_hbm.at[pl.ds(base, _CH)], idx_v)
        pltpu.async_copy(table_hbm.at[idx_v], rows_v, sem).wait()
        pltpu.sync_copy(rows_v, out_hbm.at[pl.ds(base, _CH)])
        return carry

    lax.fori_loop(0, _BPW // _CH, body, 0)


def _sc_gather(table, flat_idx):
    mesh = plsc.VectorSubcoreMesh(core_axis_name="c", subcore_axis_name="s")
    fn = functools.partial(
        pl.kernel,
        mesh=mesh,
        out_type=jax.ShapeDtypeStruct((_CNT, 64), jnp.float32),
        scratch_types=[
            pltpu.VMEM((_CH,), jnp.int32),
            pltpu.VMEM((_CH, 64), jnp.float32),
            pltpu.SemaphoreType.DMA,
        ],
        compiler_params=pltpu.CompilerParams(use_tc_tiling_on_sc=False),
    )(_sc_gather_body)
    return fn(table, flat_idx)


_NT = _CNT // 4096
_CPT = 4096 // _K


def _stats_body(xg_ref, zt_ref, s_ref, ss_ref):
    t = pl.program_id(0)
    xg = xg_ref[...].reshape(_CPT, _K, 64)
    x1 = (xg - zt_ref[...][:, None, :]).reshape(4096, 64)

    @pl.when(t == 0)
    def _init():
        s_ref[...] = jnp.zeros_like(s_ref)
        ss_ref[...] = jnp.zeros_like(ss_ref)

    s_ref[...] += jnp.sum(x1, axis=0, keepdims=True)
    ss_ref[...] += jnp.sum(x1 * x1, axis=0, keepdims=True)


def _stats(xg, zt_full):
    return pl.pallas_call(
        _stats_body,
        grid=(_NT,),
        in_specs=[
            pl.BlockSpec((4096, 64), lambda t: (t, 0)),
            pl.BlockSpec((_CPT, 64), lambda t: (t, 0)),
        ],
        out_specs=[
            pl.BlockSpec((1, 64), lambda t: (0, 0)),
            pl.BlockSpec((1, 64), lambda t: (0, 0)),
        ],
        out_shape=[
            jax.ShapeDtypeStruct((1, 64), jnp.float32),
            jax.ShapeDtypeStruct((1, 64), jnp.float32),
        ],
    )(xg, zt_full)


def _main_body(xg_ref, zt_ref, sc1_ref, of1_ref, w2t_ref, b2_ref,
               u_ref, s1_ref, g_ref):
    t = pl.program_id(0)
    xg = xg_ref[...].reshape(_CPT, _K, 64)
    x1 = (xg - zt_ref[...][:, None, :]).reshape(4096, 64)
    x1n = jnp.maximum(x1 * sc1_ref[...] + of1_ref[...], 0.0)
    x2 = jnp.dot(x1n, w2t_ref[...], preferred_element_type=jnp.float32)
    x2 = x2 + b2_ref[...]
    u_ref[...] = jnp.max(x2.reshape(_CPT, _K, 128), axis=1)

    @pl.when(t == 0)
    def _init():
        s1_ref[...] = jnp.zeros_like(s1_ref)
        g_ref[...] = jnp.zeros_like(g_ref)

    s1_ref[...] += jnp.sum(x1n, axis=0, keepdims=True)
    g_ref[...] += lax.dot_general(x1n, x1n, (((0,), (0,)), ((), ())),
                                  preferred_element_type=jnp.float32)


def _main(xg, zt_full, scale1, off1, W2, b2):
    w2t = jnp.transpose(W2, (1, 0))
    return pl.pallas_call(
        _main_body,
        grid=(_NT,),
        in_specs=[
            pl.BlockSpec((4096, 64), lambda t: (t, 0)),
            pl.BlockSpec((_CPT, 64), lambda t: (t, 0)),
            pl.BlockSpec((1, 64), lambda t: (0, 0)),
            pl.BlockSpec((1, 64), lambda t: (0, 0)),
            pl.BlockSpec((64, 128), lambda t: (0, 0)),
            pl.BlockSpec((1, 128), lambda t: (0, 0)),
        ],
        out_specs=[
            pl.BlockSpec((_CPT, 128), lambda t: (t, 0)),
            pl.BlockSpec((1, 64), lambda t: (0, 0)),
            pl.BlockSpec((64, 64), lambda t: (0, 0)),
        ],
        out_shape=[
            jax.ShapeDtypeStruct((_B * _S, 128), jnp.float32),
            jax.ShapeDtypeStruct((1, 64), jnp.float32),
            jax.ShapeDtypeStruct((64, 64), jnp.float32),
        ],
    )(xg, zt_full, scale1[None, :], off1[None, :], w2t, b2[None, :])


def _final_body(u_ref, sc2_ref, of2_ref, o_ref):
    o_ref[...] = jnp.maximum(u_ref[...] * sc2_ref[...] + of2_ref[...], 0.0)


def _final(u, scale2, off2):
    return pl.pallas_call(
        _final_body,
        grid=(_NT,),
        in_specs=[
            pl.BlockSpec((_CPT, 128), lambda t: (t, 0)),
            pl.BlockSpec((1, 128), lambda t: (0, 0)),
            pl.BlockSpec((1, 128), lambda t: (0, 0)),
        ],
        out_specs=pl.BlockSpec((_CPT, 128), lambda t: (t, 0)),
        out_shape=jax.ShapeDtypeStruct((_B * _S, 128), jnp.float32),
    )(u, scale2[None, :], off2[None, :])


def _pipeline(features, coords, W1, b1, g1, be1, W2, b2, g2, be2, gather_fn):
    centers_coords = _fps(coords)
    c_bsc = jnp.transpose(centers_coords, (0, 2, 1))
    nidx = _ball_query(c_bsc, coords)

    ft = jnp.transpose(features, (0, 2, 1))
    pt = jnp.transpose(coords, (0, 2, 1))
    yt, zt = _prep(ft, pt, c_bsc, W1, b1)

    flat_idx = (nidx + (jnp.arange(_B, dtype=jnp.int32) * _N)[:, None, None])
    flat_idx = flat_idx.reshape(_CNT)
    table = yt.reshape(_B * _N, 64)
    xg = gather_fn(table, flat_idx)

    zt_full = zt.reshape(_B * _S, 64)
    ssum, ssq = _stats(xg, zt_full)
    cnt = jnp.float32(_CNT)
    m1 = ssum[0] / cnt
    v1 = ssq[0] / cnt - m1 * m1
    scale1 = g1 / jnp.sqrt(v1 + 1e-5)
    off1 = be1 - m1 * scale1

    u, s1, g = _main(xg, zt_full, scale1, off1, W2, b2)
    mean1n = s1[0] / cnt
    m2 = jnp.dot(W2, mean1n) + b2
    e2 = (jnp.sum((jnp.dot(W2, g)) * W2, axis=1)
          + 2.0 * b2 * jnp.dot(W2, s1[0]) + cnt * b2 * b2) / cnt
    v2 = e2 - m2 * m2
    scale2 = g2 / jnp.sqrt(v2 + 1e-5)
    off2 = be2 - m2 * scale2

    o = _final(u, scale2, off2)
    out = jnp.transpose(o.reshape(_B, _S, 128), (0, 2, 1))
    return (out, centers_coords)


def kernel(features, coords, W1, b1, g1, be1, W2, b2, g2, be2):
    return _pipeline(features, coords, W1, b1, g1, be1, W2, b2, g2, be2,
                     _sc_gather)


# speedup vs baseline: 1.0791x; 1.0791x over previous
import functools

import jax
import jax.numpy as jnp
from jax import lax
from jax.experimental import pallas as pl
from jax.experimental.pallas import tpu as pltpu
from jax.experimental.pallas import tpu_sc as plsc

_B, _N, _CIN = 8, 4096, 64
_S, _K = 512, 32
_RADIUS = 0.2
_SB = 256
_FCW = 512
_CNT = _B * _S * _K


def _fps_body(x_ref, y_ref, z_ref, cx_ref, cy_ref, cz_ref, d_ref):
    ncw = _N // _FCW
    l_c = lax.broadcasted_iota(jnp.int32, (_B, _FCW), 1)
    slane = lax.broadcasted_iota(jnp.int32, (_B, _S), 1)
    for c in range(ncw):
        d_ref[:, c * _FCW:(c + 1) * _FCW] = jnp.full((_B, _FCW), 1e10,
                                                     jnp.float32)

    def step(i, carry):
        far, cxs, cys, czs = carry
        cx = jnp.zeros((_B, 1), jnp.float32)
        cy = jnp.zeros((_B, 1), jnp.float32)
        cz = jnp.zeros((_B, 1), jnp.float32)
        for c in range(ncw):
            sl = pl.ds(c * _FCW, _FCW)
            m = l_c == far - c * _FCW
            cx += jnp.sum(jnp.where(m, x_ref[:, sl], 0.0), 1, keepdims=True)
            cy += jnp.sum(jnp.where(m, y_ref[:, sl], 0.0), 1, keepdims=True)
            cz += jnp.sum(jnp.where(m, z_ref[:, sl], 0.0), 1, keepdims=True)
        sm = slane == i
        cxs = jnp.where(sm, cx, cxs)
        cys = jnp.where(sm, cy, cys)
        czs = jnp.where(sm, cz, czs)
        mx = jnp.full((_B, 1), -1.0, jnp.float32)
        for c in range(ncw):
            sl = pl.ds(c * _FCW, _FCW)
            d = ((x_ref[:, sl] - cx) ** 2 + (y_ref[:, sl] - cy) ** 2
                 + (z_ref[:, sl] - cz) ** 2)
            ds = jnp.minimum(d_ref[:, sl], d)
            d_ref[:, sl] = ds
            mx = jnp.maximum(mx, jnp.max(ds, axis=1, keepdims=True))
        far = jnp.full((_B, 1), _N, jnp.int32)
        for c in range(ncw):
            sl = pl.ds(c * _FCW, _FCW)
            idxc = jnp.min(jnp.where(d_ref[:, sl] == mx, l_c + c * _FCW, _N),
                           axis=1, keepdims=True)
            far = jnp.minimum(far, idxc)
        return far, cxs, cys, czs

    zc = jnp.zeros((_B, _S), jnp.float32)
    init = (jnp.zeros((_B, 1), jnp.int32), zc, zc, zc)
    _, cxs, cys, czs = lax.fori_loop(0, _S, step, init)
    cx_ref[...] = cxs
    cy_ref[...] = cys
    cz_ref[...] = czs


def _fps(coords):
    x = coords[:, 0, :]
    y = coords[:, 1, :]
    z = coords[:, 2, :]
    cx, cy, cz = pl.pallas_call(
        _fps_body,
        out_shape=[jax.ShapeDtypeStruct((_B, _S), jnp.float32)] * 3,
        scratch_shapes=[pltpu.VMEM((_B, _N), jnp.float32)],
    )(x, y, z)
    return jnp.stack([cx, cy, cz], axis=1)


def _ballq_body(c_ref, p_ref, o_ref):
    c = c_ref[0]
    p = p_ref[0]
    cx, cy, cz = c[:, 0:1], c[:, 1:2], c[:, 2:3]
    x, y, z = p[0:1], p[1:2], p[2:3]
    cn = (cx * cx + cy * cy) + cz * cz
    pn = (x * x + y * y) + z * z
    dp = jnp.dot(c.astype(jnp.bfloat16), p.astype(jnp.bfloat16),
                 preferred_element_type=jnp.float32)
    thr = (cn - _RADIUS * _RADIUS) + pn
    mask = (2.0 * dp >= thr).astype(jnp.int32)
    w = jnp.zeros((_SB, 128), jnp.int32)
    for j in range(_N // 128):
        w = w | (mask[:, j * 128:(j + 1) * 128] << j)
    l128 = lax.broadcasted_iota(jnp.int32, (_SB, 128), 1)
    cols = []
    for _ in range(_K):
        t = w & (-w)
        jj = lax.population_count(t - 1)
        val = jnp.where(w != 0, jj * 128 + l128, _N)
        nmin = jnp.min(val, axis=1, keepdims=True)
        cols.append(nmin)
        sh = jnp.minimum(nmin >> 7, 31)
        bit = jnp.where(nmin < _N, jnp.int32(1) << sh, 0)
        w = jnp.where(l128 == (nmin & 127), w & ~bit, w)
    nidx = jnp.concatenate(cols, axis=1)
    first = nidx[:, 0:1]
    o_ref[0] = jnp.where(nidx == _N, first, nidx)


def _ball_query(c_bsc, coords):
    grid = (_B, _S // _SB)
    return pl.pallas_call(
        _ballq_body,
        grid=grid,
        in_specs=[
            pl.BlockSpec((1, _SB, 3), lambda b, s: (b, s, 0)),
            pl.BlockSpec((1, 3, _N), lambda b, s: (b, 0, 0)),
        ],
        out_specs=pl.BlockSpec((1, _SB, _K), lambda b, s: (b, s, 0)),
        out_shape=jax.ShapeDtypeStruct((_B, _S, _K), jnp.int32),
    )(c_bsc, coords)


def _prep_body(ft_ref, pt_ref, c_ref, wf_ref, wc_ref, b1_ref, yt_ref, zt_ref):
    ft = ft_ref[0]
    pt = pt_ref[0]
    c = c_ref[0]
    wf = wf_ref[...]
    wc = wc_ref[...]
    b1 = b1_ref[...]
    yt = jnp.dot(ft, wf, preferred_element_type=jnp.float32)
    yt = yt + jnp.dot(pt, wc, preferred_element_type=jnp.float32)
    yt_ref[0] = yt + b1
    zt_ref[0] = jnp.dot(c, wc, preferred_element_type=jnp.float32)[:, :64]


def _prep(ft, pt, c_bsc, W1, b1):
    wf = jnp.zeros((_CIN, 128), jnp.float32).at[:, :64].set(
        jnp.transpose(W1[:, 3:], (1, 0)))
    wc = jnp.zeros((3, 128), jnp.float32).at[:, :64].set(
        jnp.transpose(W1[:, :3], (1, 0)))
    b1r = jnp.zeros((1, 128), jnp.float32).at[:, :64].set(b1[None, :])
    return pl.pallas_call(
        _prep_body,
        grid=(_B,),
        in_specs=[
            pl.BlockSpec((1, _N, _CIN), lambda b: (b, 0, 0)),
            pl.BlockSpec((1, _N, 3), lambda b: (b, 0, 0)),
            pl.BlockSpec((1, _S, 3), lambda b: (b, 0, 0)),
            pl.BlockSpec((_CIN, 128), lambda b: (0, 0)),
            pl.BlockSpec((3, 128), lambda b: (0, 0)),
            pl.BlockSpec((1, 128), lambda b: (0, 0)),
        ],
        out_specs=[
            pl.BlockSpec((1, _N, 128), lambda b: (b, 0, 0)),
            pl.BlockSpec((1, _S, 64), lambda b: (b, 0, 0)),
        ],
        out_shape=[
            jax.ShapeDtypeStruct((_B, _N, 128), jnp.float32),
            jax.ShapeDtypeStruct((_B, _S, 64), jnp.float32),
        ],
    )(ft, pt, c_bsc, wf, wc, b1r)


_NC, _NS = 2, 16
_NW = _NC * _NS
_BPW = _CNT // _NW
_CH = 512


def _sc_gather_body(table_hbm, idx_hbm, out_hbm, idx_v, rows_v, sem):
    wid = lax.axis_index("s") * _NC + lax.axis_index("c")

    def body(c, carry):
        base = wid * _BPW + c * _CH
        pltpu.sync_copy(idx_hbm.at[pl.ds(base, _CH)], idx_v)
        pltpu.async_copy(table_hbm.at[idx_v], rows_v, sem).wait()
        pltpu.sync_copy(rows_v, out_hbm.at[pl.ds(base, _CH)])
        return carry

    lax.fori_loop(0, _BPW // _CH, body, 0)


def _sc_gather(table, flat_idx):
    mesh = plsc.VectorSubcoreMesh(core_axis_name="c", subcore_axis_name="s")
    fn = functools.partial(
        pl.kernel,
        mesh=mesh,
        out_type=jax.ShapeDtypeStruct((_CNT, 128), jnp.float32),
        scratch_types=[
            pltpu.VMEM((_CH,), jnp.int32),
            pltpu.VMEM((_CH, 128), jnp.float32),
            pltpu.SemaphoreType.DMA,
        ],
    )(_sc_gather_body)
    return fn(table, flat_idx)


_NT = _CNT // 4096
_CPT = 4096 // _K


def _stats_body(xg_ref, zt_ref, s_ref, ss_ref):
    t = pl.program_id(0)
    xg = xg_ref[...][:, :64].reshape(_CPT, _K, 64)
    x1 = (xg - zt_ref[...][:, None, :]).reshape(4096, 64)

    @pl.when(t == 0)
    def _init():
        s_ref[...] = jnp.zeros_like(s_ref)
        ss_ref[...] = jnp.zeros_like(ss_ref)

    s_ref[...] += jnp.sum(x1, axis=0, keepdims=True)
    ss_ref[...] += jnp.sum(x1 * x1, axis=0, keepdims=True)


def _stats(xg, zt_full):
    return pl.pallas_call(
        _stats_body,
        grid=(_NT,),
        in_specs=[
            pl.BlockSpec((4096, 128), lambda t: (t, 0)),
            pl.BlockSpec((_CPT, 64), lambda t: (t, 0)),
        ],
        out_specs=[
            pl.BlockSpec((1, 64), lambda t: (0, 0)),
            pl.BlockSpec((1, 64), lambda t: (0, 0)),
        ],
        out_shape=[
            jax.ShapeDtypeStruct((1, 64), jnp.float32),
            jax.ShapeDtypeStruct((1, 64), jnp.float32),
        ],
    )(xg, zt_full)


def _main_body(xg_ref, zt_ref, sc1_ref, of1_ref, w2t_ref, b2_ref,
               u_ref, s1_ref, g_ref):
    t = pl.program_id(0)
    xg = xg_ref[...][:, :64].reshape(_CPT, _K, 64)
    x1 = (xg - zt_ref[...][:, None, :]).reshape(4096, 64)
    x1n = jnp.maximum(x1 * sc1_ref[...] + of1_ref[...], 0.0)
    x2 = jnp.dot(x1n, w2t_ref[...], preferred_element_type=jnp.float32)
    x2 = x2 + b2_ref[...]
    u_ref[...] = jnp.max(x2.reshape(_CPT, _K, 128), axis=1)

    @pl.when(t == 0)
    def _init():
        s1_ref[...] = jnp.zeros_like(s1_ref)
        g_ref[...] = jnp.zeros_like(g_ref)

    s1_ref[...] += jnp.sum(x1n, axis=0, keepdims=True)
    g_ref[...] += lax.dot_general(x1n, x1n, (((0,), (0,)), ((), ())),
                                  preferred_element_type=jnp.float32)


def _main(xg, zt_full, scale1, off1, W2, b2):
    w2t = jnp.transpose(W2, (1, 0))
    return pl.pallas_call(
        _main_body,
        grid=(_NT,),
        in_specs=[
            pl.BlockSpec((4096, 128), lambda t: (t, 0)),
            pl.BlockSpec((_CPT, 64), lambda t: (t, 0)),
            pl.BlockSpec((1, 64), lambda t: (0, 0)),
            pl.BlockSpec((1, 64), lambda t: (0, 0)),
            pl.BlockSpec((64, 128), lambda t: (0, 0)),
            pl.BlockSpec((1, 128), lambda t: (0, 0)),
        ],
        out_specs=[
            pl.BlockSpec((_CPT, 128), lambda t: (t, 0)),
            pl.BlockSpec((1, 64), lambda t: (0, 0)),
            pl.BlockSpec((64, 64), lambda t: (0, 0)),
        ],
        out_shape=[
            jax.ShapeDtypeStruct((_B * _S, 128), jnp.float32),
            jax.ShapeDtypeStruct((1, 64), jnp.float32),
            jax.ShapeDtypeStruct((64, 64), jnp.float32),
        ],
    )(xg, zt_full, scale1[None, :], off1[None, :], w2t, b2[None, :])


def _final_body(u_ref, sc2_ref, of2_ref, o_ref):
    o_ref[...] = jnp.maximum(u_ref[...] * sc2_ref[...] + of2_ref[...], 0.0)


def _final(u, scale2, off2):
    return pl.pallas_call(
        _final_body,
        grid=(_NT,),
        in_specs=[
            pl.BlockSpec((_CPT, 128), lambda t: (t, 0)),
            pl.BlockSpec((1, 128), lambda t: (0, 0)),
            pl.BlockSpec((1, 128), lambda t: (0, 0)),
        ],
        out_specs=pl.BlockSpec((_CPT, 128), lambda t: (t, 0)),
        out_shape=jax.ShapeDtypeStruct((_B * _S, 128), jnp.float32),
    )(u, scale2[None, :], off2[None, :])


def _pipeline(features, coords, W1, b1, g1, be1, W2, b2, g2, be2, gather_fn):
    centers_coords = _fps(coords)
    c_bsc = jnp.transpose(centers_coords, (0, 2, 1))
    nidx = _ball_query(c_bsc, coords)

    ft = jnp.transpose(features, (0, 2, 1))
    pt = jnp.transpose(coords, (0, 2, 1))
    yt, zt = _prep(ft, pt, c_bsc, W1, b1)

    flat_idx = (nidx + (jnp.arange(_B, dtype=jnp.int32) * _N)[:, None, None])
    flat_idx = flat_idx.reshape(_CNT)
    table = yt.reshape(_B * _N, 128)
    xg = gather_fn(table, flat_idx)

    zt_full = zt.reshape(_B * _S, 64)
    ssum, ssq = _stats(xg, zt_full)
    cnt = jnp.float32(_CNT)
    m1 = ssum[0] / cnt
    v1 = ssq[0] / cnt - m1 * m1
    scale1 = g1 / jnp.sqrt(v1 + 1e-5)
    off1 = be1 - m1 * scale1

    u, s1, g = _main(xg, zt_full, scale1, off1, W2, b2)
    mean1n = s1[0] / cnt
    m2 = jnp.dot(W2, mean1n) + b2
    e2 = (jnp.sum((jnp.dot(W2, g)) * W2, axis=1)
          + 2.0 * b2 * jnp.dot(W2, s1[0]) + cnt * b2 * b2) / cnt
    v2 = e2 - m2 * m2
    scale2 = g2 / jnp.sqrt(v2 + 1e-5)
    off2 = be2 - m2 * scale2

    o = _final(u, scale2, off2)
    out = jnp.transpose(o.reshape(_B, _S, 128), (0, 2, 1))
    return (out, centers_coords)


def kernel(features, coords, W1, b1, g1, be1, W2, b2, g2, be2):
    return _pipeline(features, coords, W1, b1, g1, be1, W2, b2, g2, be2,
                     _sc_gather)
